# baseline (speedup 1.0000x reference)
import jax
import jax.numpy as jnp
from jax import lax
from jax.experimental import pallas as pl
from jax.experimental.pallas import tpu as pltpu
from jax.experimental.pallas import tpu_sc as plsc

MARGIN = 1.0

NC = 2
NS = 16
NW = NC * NS
L = 16
TW = 128
TBLK = 25
HB = 8


def kernel(outputs, mask):
    pairs = mask.shape[0]
    assert pairs % (TW * TBLK) == 0, pairs
    ntiles = pairs // TW
    nchunks = ntiles // TBLK
    kmax = -(-nchunks // NW)
    assert kmax % 2 == 0, kmax

    nvals = outputs.shape[0]
    slice_sz = 62528
    last_sz = nvals - 15 * slice_sz

    m3 = mask.astype(jnp.int32).reshape(ntiles, TW, 2).transpose(0, 2, 1)

    mesh = plsc.VectorSubcoreMesh(
        core_axis_name="c", subcore_axis_name="s", num_cores=NC, num_subcores=NS
    )

    def body(outputs_hbm, m3_hbm, out_hbm, table_sh,
             ia0, ib0, va0, vb0, ia1, ib1, va1, vb1, acc_v,
             sem_g, sem_h, sem_i0, sem_i1):
        wid = lax.axis_index("s") * NC + lax.axis_index("c")
        sid = lax.axis_index("s")
        bufs = ((ia0, ib0, va0, vb0, sem_i0), (ia1, ib1, va1, vb1, sem_i1))

        def idx_copies(k, buf):
            ia_v, ib_v, _, _, sem_i = buf
            m = k * NW + wid
            tbase = jnp.where(m < nchunks, m, nchunks - 1) * TBLK
            ca = pltpu.make_async_copy(m3_hbm.at[pl.ds(tbase, TBLK), 0], ia_v, sem_i)
            cb = pltpu.make_async_copy(m3_hbm.at[pl.ds(tbase, TBLK), 1], ib_v, sem_i)
            return ca, cb

        c0a, c0b = idx_copies(jnp.int32(0), bufs[0])
        c0a.start()
        c0b.start()

        @pl.when(sid < 15)
        def _():
            pltpu.sync_copy(
                outputs_hbm.at[pl.ds(sid * slice_sz, slice_sz)],
                table_sh.at[pl.ds(sid * slice_sz, slice_sz)],
            )

        @pl.when(sid == 15)
        def _():
            pltpu.sync_copy(
                outputs_hbm.at[pl.ds(15 * slice_sz, last_sz)],
                table_sh.at[pl.ds(15 * slice_sz, last_sz)],
            )

        plsc.subcore_barrier()

        def super_step(k2, tot):
            for b in range(2):
                k = k2 * 2 + b
                ia_v, ib_v, va_v, vb_v, _ = bufs[b]
                m = k * NW + wid
                valid = m < nchunks

                ca, cb = idx_copies(k, bufs[b])
                ca.wait()
                cb.wait()

                def fire_h(j, _):
                    pltpu.async_copy(outputs_hbm.at[ia_v.at[j]], va_v.at[j], sem_h)
                    pltpu.async_copy(outputs_hbm.at[ib_v.at[j]], vb_v.at[j], sem_h)
                    return 0

                def fire_s(j, _):
                    pltpu.async_copy(table_sh.at[ia_v.at[j]], va_v.at[j], sem_g)
                    pltpu.async_copy(table_sh.at[ib_v.at[j]], vb_v.at[j], sem_g)
                    return 0

                lax.fori_loop(0, HB, fire_h, 0)
                lax.fori_loop(HB, TBLK, fire_s, 0)

                @pl.when(k + 1 < kmax)
                def _():
                    na, nb = idx_copies(k + 1, bufs[1 - b])
                    na.start()
                    nb.start()

                def compute_row(j, acc):
                    for g in range(TW // L):
                        va = va_v[j, pl.ds(g * L, L)]
                        vb = vb_v[j, pl.ds(g * L, L)]
                        acc = acc + jnp.maximum(MARGIN - (va - vb), 0.0)
                    return acc

                def row_s(j, acc):
                    pltpu.make_async_copy(table_sh.at[ia_v.at[j]], va_v.at[j], sem_g).wait()
                    pltpu.make_async_copy(table_sh.at[ib_v.at[j]], vb_v.at[j], sem_g).wait()
                    return compute_row(j, acc)

                def row_h(j, acc):
                    pltpu.make_async_copy(outputs_hbm.at[ia_v.at[j]], va_v.at[j], sem_h).wait()
                    pltpu.make_async_copy(outputs_hbm.at[ib_v.at[j]], vb_v.at[j], sem_h).wait()
                    return compute_row(j, acc)

                csum = lax.fori_loop(HB, TBLK, row_s, jnp.zeros((L,), jnp.float32))
                csum = lax.fori_loop(0, HB, row_h, csum)
                tot = tot + jnp.where(valid, csum, 0.0)
            return tot

        tot = lax.fori_loop(0, kmax // 2, super_step, jnp.zeros((L,), jnp.float32))
        acc_v[...] = tot
        pltpu.sync_copy(acc_v, out_hbm.at[wid])

    run = pl.kernel(
        body,
        out_type=jax.ShapeDtypeStruct((NW, L), jnp.float32),
        mesh=mesh,
        compiler_params=pltpu.CompilerParams(
            needs_layout_passes=False, use_tc_tiling_on_sc=False
        ),
        scratch_types=[
            pltpu.VMEM_SHARED((1_000_000,), jnp.float32),
            pltpu.VMEM((TBLK, TW), jnp.int32),
            pltpu.VMEM((TBLK, TW), jnp.int32),
            pltpu.VMEM((TBLK, TW), jnp.float32),
            pltpu.VMEM((TBLK, TW), jnp.float32),
            pltpu.VMEM((TBLK, TW), jnp.int32),
            pltpu.VMEM((TBLK, TW), jnp.int32),
            pltpu.VMEM((TBLK, TW), jnp.float32),
            pltpu.VMEM((TBLK, TW), jnp.float32),
            pltpu.VMEM((L,), jnp.float32),
            pltpu.SemaphoreType.DMA,
            pltpu.SemaphoreType.DMA,
            pltpu.SemaphoreType.DMA,
            pltpu.SemaphoreType.DMA,
        ],
    )
    partials = run(outputs, m3)
    return jnp.sum(partials) / jnp.float32(pairs)


# speedup vs baseline: 1.2567x; 1.2567x over previous
import jax
import jax.numpy as jnp
from jax import lax
from jax.experimental import pallas as pl
from jax.experimental.pallas import tpu as pltpu
from jax.experimental.pallas import tpu_sc as plsc

MARGIN = 1.0

NC = 2
NS = 16
NW = NC * NS
L = 16
TW = 128
TBLK = 25


def kernel(outputs, mask):
    pairs = mask.shape[0]
    assert pairs % (TW * TBLK) == 0, pairs
    ntiles = pairs // TW
    nchunks = ntiles // TBLK
    kmax = -(-nchunks // NW)
    assert kmax % 2 == 0, kmax

    nvals = outputs.shape[0]
    slice_sz = 62528
    last_sz = nvals - 15 * slice_sz

    m3 = mask.astype(jnp.int32).reshape(ntiles, TW, 2).transpose(0, 2, 1)

    mesh = plsc.VectorSubcoreMesh(
        core_axis_name="c", subcore_axis_name="s", num_cores=NC, num_subcores=NS
    )

    def body(outputs_hbm, m3_hbm, out_hbm, table_sh,
             ia0, ib0, va0, vb0, ia1, ib1, va1, vb1, acc_v,
             sem_g, sem_i0, sem_i1):
        wid = lax.axis_index("s") * NC + lax.axis_index("c")
        sid = lax.axis_index("s")
        bufs = ((ia0, ib0, va0, vb0, sem_i0), (ia1, ib1, va1, vb1, sem_i1))

        def idx_copies(k, buf):
            ia_v, ib_v, _, _, sem_i = buf
            m = k * NW + wid
            tbase = jnp.where(m < nchunks, m, nchunks - 1) * TBLK
            ca = pltpu.make_async_copy(m3_hbm.at[pl.ds(tbase, TBLK), 0], ia_v, sem_i)
            cb = pltpu.make_async_copy(m3_hbm.at[pl.ds(tbase, TBLK), 1], ib_v, sem_i)
            return ca, cb

        c0a, c0b = idx_copies(jnp.int32(0), bufs[0])
        c0a.start()
        c0b.start()

        @pl.when(sid < 15)
        def _():
            pltpu.sync_copy(
                outputs_hbm.at[pl.ds(sid * slice_sz, slice_sz)],
                table_sh.at[pl.ds(sid * slice_sz, slice_sz)],
            )

        @pl.when(sid == 15)
        def _():
            pltpu.sync_copy(
                outputs_hbm.at[pl.ds(15 * slice_sz, last_sz)],
                table_sh.at[pl.ds(15 * slice_sz, last_sz)],
            )

        plsc.subcore_barrier()

        def super_step(k2, tot):
            for b in range(2):
                k = k2 * 2 + b
                ia_v, ib_v, va_v, vb_v, _ = bufs[b]
                m = k * NW + wid
                valid = m < nchunks

                ca, cb = idx_copies(k, bufs[b])
                ca.wait()
                cb.wait()

                def fire(j, _):
                    pltpu.async_copy(table_sh.at[ia_v.at[j]], va_v.at[j], sem_g)
                    pltpu.async_copy(table_sh.at[ib_v.at[j]], vb_v.at[j], sem_g)
                    return 0

                lax.fori_loop(0, TBLK, fire, 0)

                @pl.when(k + 1 < kmax)
                def _():
                    na, nb = idx_copies(k + 1, bufs[1 - b])
                    na.start()
                    nb.start()

                def row_step(j, acc):
                    pltpu.make_async_copy(table_sh.at[ia_v.at[j]], va_v.at[j], sem_g).wait()
                    pltpu.make_async_copy(table_sh.at[ib_v.at[j]], vb_v.at[j], sem_g).wait()
                    for g in range(TW // L):
                        va = va_v[j, pl.ds(g * L, L)]
                        vb = vb_v[j, pl.ds(g * L, L)]
                        acc = acc + jnp.maximum(MARGIN - (va - vb), 0.0)
                    return acc

                csum = lax.fori_loop(0, TBLK, row_step, jnp.zeros((L,), jnp.float32))
                tot = tot + jnp.where(valid, csum, 0.0)
            return tot

        tot = lax.fori_loop(0, kmax // 2, super_step, jnp.zeros((L,), jnp.float32))
        acc_v[...] = tot
        pltpu.sync_copy(acc_v, out_hbm.at[wid])

    run = pl.kernel(
        body,
        out_type=jax.ShapeDtypeStruct((NW, L), jnp.float32),
        mesh=mesh,
        compiler_params=pltpu.CompilerParams(
            needs_layout_passes=False, use_tc_tiling_on_sc=False
        ),
        scratch_types=[
            pltpu.VMEM_SHARED((1_000_000,), jnp.float32),
            pltpu.VMEM((TBLK, TW), jnp.int32),
            pltpu.VMEM((TBLK, TW), jnp.int32),
            pltpu.VMEM((TBLK, TW), jnp.float32),
            pltpu.VMEM((TBLK, TW), jnp.float32),
            pltpu.VMEM((TBLK, TW), jnp.int32),
            pltpu.VMEM((TBLK, TW), jnp.int32),
            pltpu.VMEM((TBLK, TW), jnp.float32),
            pltpu.VMEM((TBLK, TW), jnp.float32),
            pltpu.VMEM((L,), jnp.float32),
            pltpu.SemaphoreType.DMA,
            pltpu.SemaphoreType.DMA,
            pltpu.SemaphoreType.DMA,
        ],
    )
    partials = run(outputs, m3)
    return jnp.sum(partials) / jnp.float32(pairs)
